# baseline (speedup 1.0000x reference)
import functools

import jax
import jax.numpy as jnp
from jax import lax
from jax.experimental import pallas as pl
from jax.experimental.pallas import tpu as pltpu

_VMEM_LIMIT = 48 * 1024 * 1024


def _layer_kernel(*refs, act, B, H, W, Cin, C):
    if act:
        x_ref, wr0_ref, wr1_ref, sc_ref, sh_ref, out_ref, mom_ref = refs
        xf = x_ref[...].astype(jnp.float32)
        xf = xf * sc_ref[...].reshape(1, 1, 1, Cin) + sh_ref[...].reshape(1, 1, 1, Cin)
        xb = jnp.maximum(xf, 0.0).astype(jnp.bfloat16)
    else:
        x_ref, wr0_ref, wr1_ref, out_ref, mom_ref = refs
        xb = x_ref[...]

    zrow = jnp.zeros((B, 1, W, Cin), jnp.bfloat16)
    xm = jnp.concatenate([zrow, xb[:, :-1]], axis=1)
    xp = jnp.concatenate([xb[:, 1:], zrow], axis=1)

    zcol = jnp.zeros((B, H, 1, C), jnp.float32)
    sums = []
    sqs = []
    for r, (sec, w_ref) in enumerate(((xm, wr0_ref), (xp, wr1_ref))):
        a = jnp.concatenate([xb, sec], axis=-1).reshape(B * H * W, 2 * Cin)
        t = jnp.dot(a, w_ref[...], preferred_element_type=jnp.float32)
        t = t.reshape(B, H, W, 4 * C)
        t0 = t[..., 0 * C:1 * C]
        t1 = t[..., 1 * C:2 * C]
        t2 = t[..., 2 * C:3 * C]
        t3 = t[..., 3 * C:4 * C]
        e = t1 + jnp.concatenate([zcol, t3[:, :, :-1]], axis=2)
        o = t2 + jnp.concatenate([t0[:, :, 1:], zcol], axis=2)
        y = jnp.stack([e, o], axis=3).reshape(B, H, 2 * W, C)
        y = y.astype(jnp.bfloat16)
        out_ref[:, :, r, :, :] = y
        yf = y.astype(jnp.float32).reshape(B * H * 2 * W, C)
        sums.append(jnp.sum(yf, axis=0, keepdims=True))
        sqs.append(jnp.sum(yf * yf, axis=0, keepdims=True))
    mom_ref[0, 0, :] = (sums[0] + sums[1]).reshape(C)
    mom_ref[0, 1, :] = (sqs[0] + sqs[1]).reshape(C)


def _conv_layer(x, wr0, wr1, scale, shift, B):
    N, H, W, Cin = x.shape
    C = wr0.shape[1] // 4
    G = N // B
    act = scale is not None

    in_specs = [
        pl.BlockSpec((B, H, W, Cin), lambda i: (i, 0, 0, 0)),
        pl.BlockSpec((2 * Cin, 4 * C), lambda i: (0, 0)),
        pl.BlockSpec((2 * Cin, 4 * C), lambda i: (0, 0)),
    ]
    args = [x, wr0, wr1]
    if act:
        in_specs += [
            pl.BlockSpec((1, Cin), lambda i: (0, 0)),
            pl.BlockSpec((1, Cin), lambda i: (0, 0)),
        ]
        args += [scale.reshape(1, Cin), shift.reshape(1, Cin)]

    y5, mom = pl.pallas_call(
        functools.partial(_layer_kernel, act=act, B=B, H=H, W=W, Cin=Cin, C=C),
        out_shape=(
            jax.ShapeDtypeStruct((N, H, 2, 2 * W, C), jnp.bfloat16),
            jax.ShapeDtypeStruct((G, 2, C), jnp.float32),
        ),
        grid_spec=pltpu.PrefetchScalarGridSpec(
            num_scalar_prefetch=0,
            grid=(G,),
            in_specs=in_specs,
            out_specs=(
                pl.BlockSpec((B, H, 2, 2 * W, C), lambda i: (i, 0, 0, 0, 0)),
                pl.BlockSpec((1, 2, C), lambda i: (i, 0, 0)),
            ),
        ),
        compiler_params=pltpu.CompilerParams(
            dimension_semantics=("parallel",),
            vmem_limit_bytes=_VMEM_LIMIT,
        ),
    )(*args)
    return y5, mom


def _final_kernel(x_ref, w_ref, b_ref, sc_ref, sh_ref, p_ref, out_ref,
                  *, H, W, Cin, RC):
    xf = x_ref[0].astype(jnp.float32)
    xf = xf * sc_ref[...].reshape(1, 1, Cin) + sh_ref[...].reshape(1, 1, Cin)
    xb = jnp.maximum(xf, 0.0).astype(jnp.bfloat16)

    zrow = jnp.zeros((1, W, Cin), jnp.bfloat16)
    nch = H // RC
    KR = RC + 2
    L = KR * W
    jj = lax.broadcasted_iota(jnp.int32, (3, L), 1) % W
    first = jj == 0
    last = jj == W - 1
    z1 = jnp.zeros((3, 1), jnp.float32)
    zW = jnp.zeros((3, W), jnp.float32)
    bias = b_ref[...].reshape(3, 1, 1)
    perm = p_ref[...]

    for ci in range(nch):
        r0 = ci * RC
        if ci == 0:
            xch = jnp.concatenate([zrow, xb[:KR - 1]], axis=0)
        elif ci == nch - 1:
            xch = jnp.concatenate([xb[r0 - 1:], zrow], axis=0)
        else:
            xch = xb[r0 - 1:r0 + KR - 1]
        t = lax.dot_general(w_ref[...], xch.reshape(KR * W, Cin),
                            (((0,), (1,)), ((), ())),
                            preferred_element_type=jnp.float32)
        t = t.reshape(16, 3, L)
        ecol = []
        ocol = []
        for kh in range(4):
            t0, t1, t2, t3 = (t[4 * kh + i] for i in range(4))
            sh_m = jnp.concatenate([z1, t3[:, :-1]], axis=1)
            ecol.append(t1 + jnp.where(first, 0.0, sh_m))
            sh_p = jnp.concatenate([t0[:, 1:], z1], axis=1)
            ocol.append(t2 + jnp.where(last, 0.0, sh_p))
        rows_e = []
        rows_o = []
        for col, rows in ((ecol, rows_e), (ocol, rows_o)):
            v0 = col[1] + jnp.concatenate([zW, col[3][:, :-W]], axis=1)
            v1 = col[2] + jnp.concatenate([col[0][:, W:], zW], axis=1)
            for k in range(1, KR - 1):
                rows.append(v0[:, k * W:(k + 1) * W])
                rows.append(v1[:, k * W:(k + 1) * W])
        ge = jnp.stack(rows_e, axis=1)
        go = jnp.stack(rows_o, axis=1)
        g = jnp.concatenate([ge, go], axis=-1).reshape(3 * 2 * RC, 2 * W)
        out = jnp.dot(g, perm, preferred_element_type=jnp.float32)
        out = jnp.tanh(out.reshape(3, 2 * RC, 2 * W) + bias)
        out_ref[0, :, 2 * r0:2 * (r0 + RC), :] = out


def _final_layer(x, bm4, b4, scale, shift):
    N, H, W, Cin = x.shape
    RC = 4
    src = jnp.arange(2 * W)
    dst = (src % W) * 2 + src // W
    perm = jnp.zeros((2 * W, 2 * W), jnp.float32).at[src, dst].set(1.0)
    out = pl.pallas_call(
        functools.partial(_final_kernel, H=H, W=W, Cin=Cin, RC=RC),
        out_shape=jax.ShapeDtypeStruct((N, 3, 2 * H, 2 * W), jnp.float32),
        grid_spec=pltpu.PrefetchScalarGridSpec(
            num_scalar_prefetch=0,
            grid=(N,),
            in_specs=[
                pl.BlockSpec((1, H, W, Cin), lambda i: (i, 0, 0, 0)),
                pl.BlockSpec((Cin, 48), lambda i: (0, 0)),
                pl.BlockSpec((3, 1), lambda i: (0, 0)),
                pl.BlockSpec((1, Cin), lambda i: (0, 0)),
                pl.BlockSpec((1, Cin), lambda i: (0, 0)),
                pl.BlockSpec((2 * W, 2 * W), lambda i: (0, 0)),
            ],
            out_specs=pl.BlockSpec((1, 3, 2 * H, 2 * W), lambda i: (i, 0, 0, 0)),
        ),
        compiler_params=pltpu.CompilerParams(
            dimension_semantics=("parallel",),
            vmem_limit_bytes=_VMEM_LIMIT,
        ),
    )(x, bm4, b4.reshape(3, 1).astype(jnp.float32),
      scale.reshape(1, Cin), shift.reshape(1, Cin), perm)
    return out


def _split_weights(bm):
    cin = bm.shape[0]
    c4 = bm.shape[1] // 4
    w = [bm[:, k * c4:(k + 1) * c4] for k in range(4)]
    wr0 = jnp.concatenate([w[1], w[3]], axis=0)
    wr1 = jnp.concatenate([w[2], w[0]], axis=0)
    return wr0, wr1


def _bn_stats(mom, count, gamma, beta):
    tot = jnp.sum(mom.astype(jnp.float32), axis=0)
    mean = tot[0] / count
    msq = tot[1] / count
    var = jnp.maximum(msq - jnp.square(mean), 0.0)
    scale = gamma * lax.rsqrt(var + 1e-5)
    shift = beta - mean * scale
    return scale, shift


def _pick_b(n, pref):
    b = min(pref, n)
    while n % b:
        b -= 1
    return b


@jax.jit
def _forward(x, params):
    N = x.shape[0]
    xb = jnp.transpose(x, (0, 2, 3, 1)).astype(jnp.bfloat16)

    scale = shift = None
    prefs = (32, 8, 4, 2)
    for li in range(4):
        bm, gamma, beta = params[li]
        wr0, wr1 = _split_weights(bm)
        B = _pick_b(N, prefs[li])
        y5, mom = _conv_layer(xb, wr0, wr1, scale, shift, B)
        Nn, H, _, W2, C = y5.shape
        count = jnp.float32(Nn * 2 * H * W2)
        scale, shift = _bn_stats(mom, count, gamma, beta)
        xb = y5.reshape(Nn, 2 * H, W2, C)

    bm4, b4 = params[4]
    return _final_layer(xb, bm4, b4, scale, shift)


def kernel(x, bm_0, b_0, gamma_0, beta_0, bm_1, b_1, gamma_1, beta_1,
           bm_2, b_2, gamma_2, beta_2, bm_3, b_3, gamma_3, beta_3,
           bm_4, b_4):
    params = [
        (bm_0, gamma_0, beta_0),
        (bm_1, gamma_1, beta_1),
        (bm_2, gamma_2, beta_2),
        (bm_3, gamma_3, beta_3),
        (bm_4, b_4),
    ]
    return _forward(x, params)


# speedup vs baseline: 1.1008x; 1.1008x over previous
import functools

import jax
import jax.numpy as jnp
from jax import lax
from jax.experimental import pallas as pl
from jax.experimental.pallas import tpu as pltpu

_VMEM_LIMIT = 48 * 1024 * 1024


def _layer_kernel(*refs, act, B, H, W, Cin, C, pair):
    if act:
        x_ref, wr0_ref, wr1_ref, sc_ref, sh_ref, out_ref, mom_ref = refs
        xf = x_ref[...].astype(jnp.float32)
        xf = xf * sc_ref[...].reshape(1, 1, 1, Cin) + sh_ref[...].reshape(1, 1, 1, Cin)
        xb = jnp.maximum(xf, 0.0).astype(jnp.bfloat16)
    else:
        x_ref, wr0_ref, wr1_ref, out_ref, mom_ref = refs
        xb = x_ref[...]

    zrow = jnp.zeros((B, 1, W, Cin), jnp.bfloat16)
    xm = jnp.concatenate([zrow, xb[:, :-1]], axis=1)
    xp = jnp.concatenate([xb[:, 1:], zrow], axis=1)

    zcol = jnp.zeros((B, H, 1, C), jnp.float32)
    sums = []
    sqs = []
    for r, (sec, w_ref) in enumerate(((xm, wr0_ref), (xp, wr1_ref))):
        a = jnp.concatenate([xb, sec], axis=-1).reshape(B * H * W, 2 * Cin)
        t = jnp.dot(a, w_ref[...], preferred_element_type=jnp.float32)
        t = t.reshape(B, H, W, 4 * C)
        t0 = t[..., 0 * C:1 * C]
        t1 = t[..., 1 * C:2 * C]
        t2 = t[..., 2 * C:3 * C]
        t3 = t[..., 3 * C:4 * C]
        e = t1 + jnp.concatenate([zcol, t3[:, :, :-1]], axis=2)
        o = t2 + jnp.concatenate([t0[:, :, 1:], zcol], axis=2)
        if pair:
            y = jnp.concatenate([e, o], axis=-1).astype(jnp.bfloat16)
            out_ref[:, :, r, :, :] = y
            yf = y.astype(jnp.float32).reshape(B * H * W, 2 * C)
            s2 = jnp.sum(yf, axis=0, keepdims=True)
            q2 = jnp.sum(yf * yf, axis=0, keepdims=True)
            sums.append(s2[:, :C] + s2[:, C:])
            sqs.append(q2[:, :C] + q2[:, C:])
        else:
            y = jnp.stack([e, o], axis=3).reshape(B, H, 2 * W, C)
            y = y.astype(jnp.bfloat16)
            out_ref[:, :, r, :, :] = y
            yf = y.astype(jnp.float32).reshape(B * H * 2 * W, C)
            sums.append(jnp.sum(yf, axis=0, keepdims=True))
            sqs.append(jnp.sum(yf * yf, axis=0, keepdims=True))
    mom_ref[0, 0, :] = (sums[0] + sums[1]).reshape(C)
    mom_ref[0, 1, :] = (sqs[0] + sqs[1]).reshape(C)


def _conv_layer(x, wr0, wr1, scale, shift, B, pair=False):
    N, H, W, Cin = x.shape
    C = wr0.shape[1] // 4
    G = N // B
    act = scale is not None

    in_specs = [
        pl.BlockSpec((B, H, W, Cin), lambda i: (i, 0, 0, 0)),
        pl.BlockSpec((2 * Cin, 4 * C), lambda i: (0, 0)),
        pl.BlockSpec((2 * Cin, 4 * C), lambda i: (0, 0)),
    ]
    args = [x, wr0, wr1]
    if act:
        in_specs += [
            pl.BlockSpec((1, Cin), lambda i: (0, 0)),
            pl.BlockSpec((1, Cin), lambda i: (0, 0)),
        ]
        args += [scale.reshape(1, Cin), shift.reshape(1, Cin)]

    if pair:
        oshape = (N, H, 2, W, 2 * C)
        oblock = (B, H, 2, W, 2 * C)
    else:
        oshape = (N, H, 2, 2 * W, C)
        oblock = (B, H, 2, 2 * W, C)

    y5, mom = pl.pallas_call(
        functools.partial(_layer_kernel, act=act, B=B, H=H, W=W, Cin=Cin,
                          C=C, pair=pair),
        out_shape=(
            jax.ShapeDtypeStruct(oshape, jnp.bfloat16),
            jax.ShapeDtypeStruct((G, 2, C), jnp.float32),
        ),
        grid_spec=pltpu.PrefetchScalarGridSpec(
            num_scalar_prefetch=0,
            grid=(G,),
            in_specs=in_specs,
            out_specs=(
                pl.BlockSpec(oblock, lambda i: (i, 0, 0, 0, 0)),
                pl.BlockSpec((1, 2, C), lambda i: (i, 0, 0)),
            ),
        ),
        compiler_params=pltpu.CompilerParams(
            dimension_semantics=("parallel",),
            vmem_limit_bytes=_VMEM_LIMIT,
        ),
    )(*args)
    return y5, mom


def _final_kernel(x_ref, w_ref, b_ref, sc_ref, sh_ref, p_ref, out_ref,
                  xp_ref, *, H, Wp, Cin, RC):
    k = pl.program_id(1)

    @pl.when(k == 0)
    def _prologue():
        xf = x_ref[0].astype(jnp.float32)
        xf = xf * sc_ref[...].reshape(1, 1, 2 * Cin) \
            + sh_ref[...].reshape(1, 1, 2 * Cin)
        xp_ref[1:H + 1] = jnp.maximum(xf, 0.0).astype(jnp.bfloat16)
        z = jnp.zeros((1, Wp, 2 * Cin), jnp.bfloat16)
        xp_ref[0:1] = z
        xp_ref[H + 1:H + 2] = z

    KR = RC + 2
    L = KR * Wp
    xch = xp_ref[pl.ds(k * RC, KR)]
    t = lax.dot_general(w_ref[...], xch.reshape(L, 2 * Cin),
                        (((0,), (1,)), ((), ())),
                        preferred_element_type=jnp.float32)
    t = t.reshape(2, 16, 3, L)

    vv = lax.broadcasted_iota(jnp.int32, (3, L), 1) % Wp
    first = vv == 0
    last = vv == Wp - 1
    z1 = jnp.zeros((3, 1), jnp.float32)
    zW = jnp.zeros((3, Wp), jnp.float32)
    cls = [[], [], [], []]
    for kh in range(4):
        t_ = [[t[s, 4 * kh + i] for i in range(4)] for s in range(2)]
        shm = jnp.concatenate([z1, t_[1][3][:, :-1]], axis=1)
        cls[0].append(t_[0][1] + jnp.where(first, 0.0, shm))
        cls[1].append(t_[0][2] + t_[1][0])
        cls[2].append(t_[1][1] + t_[0][3])
        shp = jnp.concatenate([t_[0][0][:, 1:], z1], axis=1)
        cls[3].append(t_[1][2] + jnp.where(last, 0.0, shp))

    rows = []
    vq = []
    for q in range(4):
        c_ = cls[q]
        v0 = c_[1] + jnp.concatenate([zW, c_[3][:, :-Wp]], axis=1)
        v1 = c_[2] + jnp.concatenate([c_[0][:, Wp:], zW], axis=1)
        vq.append((v0, v1))
    for kk in range(1, KR - 1):
        for r in range(2):
            rows.append(jnp.concatenate(
                [vq[q][r][:, kk * Wp:(kk + 1) * Wp] for q in range(4)],
                axis=-1))
    g = jnp.stack(rows, axis=1).reshape(3 * 2 * RC, 4 * Wp)
    out = jnp.dot(g, p_ref[...], preferred_element_type=jnp.float32)
    out = jnp.tanh(out.reshape(3, 2 * RC, 4 * Wp) + b_ref[...].reshape(3, 1, 1))
    out_ref[0] = out


def _final_layer(x, bm4, b4, scale, shift):
    N, H, Wp, Cin2 = x.shape
    Cin = Cin2 // 2
    RC = 4
    N2 = N // 2
    NCH = H // RC
    z = jnp.zeros((Cin, 48), bm4.dtype)
    w2 = jnp.concatenate(
        [jnp.concatenate([bm4, z], axis=1),
         jnp.concatenate([z, bm4], axis=1)], axis=0)
    src = jnp.arange(4 * Wp)
    dst = (src % Wp) * 4 + src // Wp
    perm = jnp.zeros((4 * Wp, 4 * Wp), jnp.float32).at[src, dst].set(1.0)
    sc2 = jnp.concatenate([scale, scale]).reshape(1, 2 * Cin)
    sh2 = jnp.concatenate([shift, shift]).reshape(1, 2 * Cin)
    out = pl.pallas_call(
        functools.partial(_final_kernel, H=H, Wp=Wp, Cin=Cin, RC=RC),
        out_shape=jax.ShapeDtypeStruct((N, 3, 2 * H, 4 * Wp), jnp.float32),
        grid_spec=pltpu.PrefetchScalarGridSpec(
            num_scalar_prefetch=0,
            grid=(N, NCH),
            in_specs=[
                pl.BlockSpec((1, H, Wp, 2 * Cin), lambda i, k: (i, 0, 0, 0)),
                pl.BlockSpec((2 * Cin, 96), lambda i, k: (0, 0)),
                pl.BlockSpec((3, 1), lambda i, k: (0, 0)),
                pl.BlockSpec((1, 2 * Cin), lambda i, k: (0, 0)),
                pl.BlockSpec((1, 2 * Cin), lambda i, k: (0, 0)),
                pl.BlockSpec((4 * Wp, 4 * Wp), lambda i, k: (0, 0)),
            ],
            out_specs=pl.BlockSpec(
                (1, 3, 2 * RC, 4 * Wp), lambda i, k: (i, 0, k, 0)),
            scratch_shapes=[pltpu.VMEM((H + 2, Wp, 2 * Cin), jnp.bfloat16)],
        ),
        compiler_params=pltpu.CompilerParams(
            dimension_semantics=("parallel", "arbitrary"),
            vmem_limit_bytes=_VMEM_LIMIT,
        ),
    )(x, w2, b4.reshape(3, 1).astype(jnp.float32), sc2, sh2, perm)
    return out


def _split_weights(bm):
    cin = bm.shape[0]
    c4 = bm.shape[1] // 4
    w = [bm[:, k * c4:(k + 1) * c4] for k in range(4)]
    wr0 = jnp.concatenate([w[1], w[3]], axis=0)
    wr1 = jnp.concatenate([w[2], w[0]], axis=0)
    return wr0, wr1


def _bn_stats(mom, count, gamma, beta):
    tot = jnp.sum(mom.astype(jnp.float32), axis=0)
    mean = tot[0] / count
    msq = tot[1] / count
    var = jnp.maximum(msq - jnp.square(mean), 0.0)
    scale = gamma * lax.rsqrt(var + 1e-5)
    shift = beta - mean * scale
    return scale, shift


def _pick_b(n, pref):
    b = min(pref, n)
    while b > 1 and (n % b or (n // b) % 2):
        b -= 1
    return b


@jax.jit
def _forward(x, params):
    N = x.shape[0]
    xb = jnp.transpose(x, (0, 2, 3, 1)).astype(jnp.bfloat16)

    scale = shift = None
    prefs = (32, 8, 4, 2)
    for li in range(4):
        bm, gamma, beta = params[li]
        wr0, wr1 = _split_weights(bm)
        B = _pick_b(N, prefs[li])
        pair = li == 3
        y5, mom = _conv_layer(xb, wr0, wr1, scale, shift, B, pair=pair)
        Nn, H, _, W2, C = y5.shape
        npix = Nn * 2 * H * W2 * (2 if pair else 1)
        scale, shift = _bn_stats(mom, jnp.float32(npix), gamma, beta)
        xb = y5.reshape(Nn, 2 * H, W2, C)

    bm4, b4 = params[4]
    return _final_layer(xb, bm4, b4, scale, shift)


def kernel(x, bm_0, b_0, gamma_0, beta_0, bm_1, b_1, gamma_1, beta_1,
           bm_2, b_2, gamma_2, beta_2, bm_3, b_3, gamma_3, beta_3,
           bm_4, b_4):
    params = [
        (bm_0, gamma_0, beta_0),
        (bm_1, gamma_1, beta_1),
        (bm_2, gamma_2, beta_2),
        (bm_3, gamma_3, beta_3),
        (bm_4, b_4),
    ]
    return _forward(x, params)


# speedup vs baseline: 3.2162x; 2.9217x over previous
import functools

import jax
import jax.numpy as jnp
from jax import lax
from jax.experimental import pallas as pl
from jax.experimental.pallas import tpu as pltpu

_VMEM_LIMIT = 48 * 1024 * 1024


def _layer_kernel(*refs, act, B, H, W, Cin, C, pair):
    if act:
        x_ref, wr0_ref, wr1_ref, sc_ref, sh_ref, out_ref, mom_ref = refs
        xf = x_ref[...].astype(jnp.float32)
        xf = xf * sc_ref[...].reshape(1, 1, 1, Cin) + sh_ref[...].reshape(1, 1, 1, Cin)
        xb = jnp.maximum(xf, 0.0).astype(jnp.bfloat16)
    else:
        x_ref, wr0_ref, wr1_ref, out_ref, mom_ref = refs
        xb = x_ref[...]

    zrow = jnp.zeros((B, 1, W, Cin), jnp.bfloat16)
    xm = jnp.concatenate([zrow, xb[:, :-1]], axis=1)
    xp = jnp.concatenate([xb[:, 1:], zrow], axis=1)

    zcol = jnp.zeros((B, H, 1, C), jnp.float32)
    sums = []
    sqs = []
    for r, (sec, w_ref) in enumerate(((xm, wr0_ref), (xp, wr1_ref))):
        a = jnp.concatenate([xb, sec], axis=-1).reshape(B * H * W, 2 * Cin)
        t = jnp.dot(a, w_ref[...], preferred_element_type=jnp.float32)
        t = t.reshape(B, H, W, 4 * C)
        t0 = t[..., 0 * C:1 * C]
        t1 = t[..., 1 * C:2 * C]
        t2 = t[..., 2 * C:3 * C]
        t3 = t[..., 3 * C:4 * C]
        e = t1 + jnp.concatenate([zcol, t3[:, :, :-1]], axis=2)
        o = t2 + jnp.concatenate([t0[:, :, 1:], zcol], axis=2)
        if pair:
            y = jnp.concatenate([e, o], axis=-1).astype(jnp.bfloat16)
            out_ref[:, :, r, :, :] = y
            yf = y.astype(jnp.float32).reshape(B * H * W, 2 * C)
            s2 = jnp.sum(yf, axis=0, keepdims=True)
            q2 = jnp.sum(yf * yf, axis=0, keepdims=True)
            sums.append(s2[:, :C] + s2[:, C:])
            sqs.append(q2[:, :C] + q2[:, C:])
        else:
            y = jnp.stack([e, o], axis=3).reshape(B, H, 2 * W, C)
            y = y.astype(jnp.bfloat16)
            out_ref[:, :, r, :, :] = y
            yf = y.astype(jnp.float32).reshape(B * H * 2 * W, C)
            sums.append(jnp.sum(yf, axis=0, keepdims=True))
            sqs.append(jnp.sum(yf * yf, axis=0, keepdims=True))
    mom_ref[0, 0, :] = (sums[0] + sums[1]).reshape(C)
    mom_ref[0, 1, :] = (sqs[0] + sqs[1]).reshape(C)


def _conv_layer(x, wr0, wr1, scale, shift, B, pair=False):
    N, H, W, Cin = x.shape
    C = wr0.shape[1] // 4
    G = N // B
    act = scale is not None

    in_specs = [
        pl.BlockSpec((B, H, W, Cin), lambda i: (i, 0, 0, 0)),
        pl.BlockSpec((2 * Cin, 4 * C), lambda i: (0, 0)),
        pl.BlockSpec((2 * Cin, 4 * C), lambda i: (0, 0)),
    ]
    args = [x, wr0, wr1]
    if act:
        in_specs += [
            pl.BlockSpec((1, Cin), lambda i: (0, 0)),
            pl.BlockSpec((1, Cin), lambda i: (0, 0)),
        ]
        args += [scale.reshape(1, Cin), shift.reshape(1, Cin)]

    if pair:
        oshape = (N, H, 2, W, 2 * C)
        oblock = (B, H, 2, W, 2 * C)
    else:
        oshape = (N, H, 2, 2 * W, C)
        oblock = (B, H, 2, 2 * W, C)

    y5, mom = pl.pallas_call(
        functools.partial(_layer_kernel, act=act, B=B, H=H, W=W, Cin=Cin,
                          C=C, pair=pair),
        out_shape=(
            jax.ShapeDtypeStruct(oshape, jnp.bfloat16),
            jax.ShapeDtypeStruct((G, 2, C), jnp.float32),
        ),
        grid_spec=pltpu.PrefetchScalarGridSpec(
            num_scalar_prefetch=0,
            grid=(G,),
            in_specs=in_specs,
            out_specs=(
                pl.BlockSpec(oblock, lambda i: (i, 0, 0, 0, 0)),
                pl.BlockSpec((1, 2, C), lambda i: (i, 0, 0)),
            ),
        ),
        compiler_params=pltpu.CompilerParams(
            dimension_semantics=("parallel",),
            vmem_limit_bytes=_VMEM_LIMIT,
        ),
    )(*args)
    return y5, mom


def _final_kernel(x_ref, w_ref, b_ref, sc_ref, sh_ref, out_ref, *, H, Wp, Cin):
    L = H * Wp
    xf = x_ref[0].astype(jnp.float32)
    xf = xf * sc_ref[...].reshape(1, 1, 2 * Cin) \
        + sh_ref[...].reshape(1, 1, 2 * Cin)
    xb = jnp.maximum(xf, 0.0).astype(jnp.bfloat16)
    t = lax.dot_general(w_ref[...], xb.reshape(L, 2 * Cin),
                        (((0,), (1,)), ((), ())),
                        preferred_element_type=jnp.float32)
    t = t.reshape(2, 16, 3, L)

    vv = lax.broadcasted_iota(jnp.int32, (3, L), 1) % Wp
    first = vv == 0
    last = vv == Wp - 1
    z1 = jnp.zeros((3, 1), jnp.float32)
    zW = jnp.zeros((3, Wp), jnp.float32)
    cls = [[], [], [], []]
    for kh in range(4):
        t_ = [[t[s, 4 * kh + i] for i in range(4)] for s in range(2)]
        shm = jnp.concatenate([z1, t_[1][3][:, :-1]], axis=1)
        cls[0].append(t_[0][1] + jnp.where(first, 0.0, shm))
        cls[1].append(t_[0][2] + t_[1][0])
        cls[2].append(t_[1][1] + t_[0][3])
        shp = jnp.concatenate([t_[0][0][:, 1:], z1], axis=1)
        cls[3].append(t_[1][2] + jnp.where(last, 0.0, shp))

    bias = b_ref[...].reshape(3, 1)
    for q in range(4):
        c_ = cls[q]
        v0 = c_[1] + jnp.concatenate([zW, c_[3][:, :-Wp]], axis=1)
        v1 = c_[2] + jnp.concatenate([c_[0][:, Wp:], zW], axis=1)
        out_ref[0, q] = jnp.tanh(v0 + bias)
        out_ref[0, 4 + q] = jnp.tanh(v1 + bias)


def _final_kernel_unused(x_ref, w_ref, b_ref, sc_ref, sh_ref, p_ref, out_ref,
                         xp_ref, *, H, Wp, Cin, RC):
    k = pl.program_id(1)

    @pl.when(k == 0)
    def _prologue():
        xf = x_ref[0].astype(jnp.float32)
        xf = xf * sc_ref[...].reshape(1, 1, 2 * Cin) \
            + sh_ref[...].reshape(1, 1, 2 * Cin)
        xp_ref[1:H + 1] = jnp.maximum(xf, 0.0).astype(jnp.bfloat16)
        z = jnp.zeros((1, Wp, 2 * Cin), jnp.bfloat16)
        xp_ref[0:1] = z
        xp_ref[H + 1:H + 2] = z

    KR = RC + 2
    L = KR * Wp
    xch = xp_ref[pl.ds(k * RC, KR)]
    t = lax.dot_general(w_ref[...], xch.reshape(L, 2 * Cin),
                        (((0,), (1,)), ((), ())),
                        preferred_element_type=jnp.float32)
    t = t.reshape(2, 16, 3, L)

    vv = lax.broadcasted_iota(jnp.int32, (3, L), 1) % Wp
    first = vv == 0
    last = vv == Wp - 1
    z1 = jnp.zeros((3, 1), jnp.float32)
    zW = jnp.zeros((3, Wp), jnp.float32)
    cls = [[], [], [], []]
    for kh in range(4):
        t_ = [[t[s, 4 * kh + i] for i in range(4)] for s in range(2)]
        shm = jnp.concatenate([z1, t_[1][3][:, :-1]], axis=1)
        cls[0].append(t_[0][1] + jnp.where(first, 0.0, shm))
        cls[1].append(t_[0][2] + t_[1][0])
        cls[2].append(t_[1][1] + t_[0][3])
        shp = jnp.concatenate([t_[0][0][:, 1:], z1], axis=1)
        cls[3].append(t_[1][2] + jnp.where(last, 0.0, shp))

    rows = []
    vq = []
    for q in range(4):
        c_ = cls[q]
        v0 = c_[1] + jnp.concatenate([zW, c_[3][:, :-Wp]], axis=1)
        v1 = c_[2] + jnp.concatenate([c_[0][:, Wp:], zW], axis=1)
        vq.append((v0, v1))
    for kk in range(1, KR - 1):
        for r in range(2):
            rows.append(jnp.concatenate(
                [vq[q][r][:, kk * Wp:(kk + 1) * Wp] for q in range(4)],
                axis=-1))
    g = jnp.stack(rows, axis=1).reshape(3 * 2 * RC, 4 * Wp)
    out = jnp.dot(g, p_ref[...], preferred_element_type=jnp.float32)
    out = jnp.tanh(out.reshape(3, 2 * RC, 4 * Wp) + b_ref[...].reshape(3, 1, 1))
    out_ref[0] = out


def _final_layer(x, bm4, b4, scale, shift):
    N, H, Wp, Cin2 = x.shape
    Cin = Cin2 // 2
    z = jnp.zeros((Cin, 48), bm4.dtype)
    w2 = jnp.concatenate(
        [jnp.concatenate([bm4, z], axis=1),
         jnp.concatenate([z, bm4], axis=1)], axis=0)
    sc2 = jnp.concatenate([scale, scale]).reshape(1, 2 * Cin)
    sh2 = jnp.concatenate([shift, shift]).reshape(1, 2 * Cin)
    o = pl.pallas_call(
        functools.partial(_final_kernel, H=H, Wp=Wp, Cin=Cin),
        out_shape=jax.ShapeDtypeStruct((N, 8, 3, H * Wp), jnp.float32),
        grid_spec=pltpu.PrefetchScalarGridSpec(
            num_scalar_prefetch=0,
            grid=(N,),
            in_specs=[
                pl.BlockSpec((1, H, Wp, 2 * Cin), lambda i: (i, 0, 0, 0)),
                pl.BlockSpec((2 * Cin, 96), lambda i: (0, 0)),
                pl.BlockSpec((3, 1), lambda i: (0, 0)),
                pl.BlockSpec((1, 2 * Cin), lambda i: (0, 0)),
                pl.BlockSpec((1, 2 * Cin), lambda i: (0, 0)),
            ],
            out_specs=pl.BlockSpec((1, 8, 3, H * Wp), lambda i: (i, 0, 0, 0)),
        ),
        compiler_params=pltpu.CompilerParams(
            dimension_semantics=("parallel",),
            vmem_limit_bytes=_VMEM_LIMIT,
        ),
    )(x, w2, b4.reshape(3, 1).astype(jnp.float32), sc2, sh2)
    o = o.reshape(N, 2, 4, 3, H, Wp)
    o = jnp.transpose(o, (0, 3, 4, 1, 5, 2))
    return o.reshape(N, 3, 2 * H, 4 * Wp)


def _split_weights(bm):
    cin = bm.shape[0]
    c4 = bm.shape[1] // 4
    w = [bm[:, k * c4:(k + 1) * c4] for k in range(4)]
    wr0 = jnp.concatenate([w[1], w[3]], axis=0)
    wr1 = jnp.concatenate([w[2], w[0]], axis=0)
    return wr0, wr1


def _bn_stats(mom, count, gamma, beta):
    tot = jnp.sum(mom.astype(jnp.float32), axis=0)
    mean = tot[0] / count
    msq = tot[1] / count
    var = jnp.maximum(msq - jnp.square(mean), 0.0)
    scale = gamma * lax.rsqrt(var + 1e-5)
    shift = beta - mean * scale
    return scale, shift


def _pick_b(n, pref):
    b = min(pref, n)
    while b > 1 and (n % b or (n // b) % 2):
        b -= 1
    return b


@jax.jit
def _forward(x, params):
    N = x.shape[0]
    xb = jnp.transpose(x, (0, 2, 3, 1)).astype(jnp.bfloat16)

    scale = shift = None
    prefs = (32, 8, 4, 2)
    for li in range(4):
        bm, gamma, beta = params[li]
        wr0, wr1 = _split_weights(bm)
        B = _pick_b(N, prefs[li])
        pair = li == 3
        y5, mom = _conv_layer(xb, wr0, wr1, scale, shift, B, pair=pair)
        Nn, H, _, W2, C = y5.shape
        npix = Nn * 2 * H * W2 * (2 if pair else 1)
        scale, shift = _bn_stats(mom, jnp.float32(npix), gamma, beta)
        xb = y5.reshape(Nn, 2 * H, W2, C)

    bm4, b4 = params[4]
    return _final_layer(xb, bm4, b4, scale, shift)


def kernel(x, bm_0, b_0, gamma_0, beta_0, bm_1, b_1, gamma_1, beta_1,
           bm_2, b_2, gamma_2, beta_2, bm_3, b_3, gamma_3, beta_3,
           bm_4, b_4):
    params = [
        (bm_0, gamma_0, beta_0),
        (bm_1, gamma_1, beta_1),
        (bm_2, gamma_2, beta_2),
        (bm_3, gamma_3, beta_3),
        (bm_4, b_4),
    ]
    return _forward(x, params)


# speedup vs baseline: 3.2541x; 1.0118x over previous
import functools

import jax
import jax.numpy as jnp
from jax import lax
from jax.experimental import pallas as pl
from jax.experimental.pallas import tpu as pltpu

_VMEM_LIMIT = 48 * 1024 * 1024


def _layer_kernel(*refs, act, B, H, W, Cin, C, pair):
    if act:
        x_ref, wr0_ref, wr1_ref, sc_ref, sh_ref, out_ref, mom_ref = refs
        xf = x_ref[...].astype(jnp.float32)
        xf = xf * sc_ref[...].reshape(1, 1, 1, Cin) + sh_ref[...].reshape(1, 1, 1, Cin)
        xb = jnp.maximum(xf, 0.0).astype(jnp.bfloat16)
    else:
        x_ref, wr0_ref, wr1_ref, out_ref, mom_ref = refs
        xb = x_ref[...]

    zrow = jnp.zeros((B, 1, W, Cin), jnp.bfloat16)
    xm = jnp.concatenate([zrow, xb[:, :-1]], axis=1)
    xp = jnp.concatenate([xb[:, 1:], zrow], axis=1)

    zcol = jnp.zeros((B, H, 1, C), jnp.float32)
    sums = []
    sqs = []
    for r, (sec, w_ref) in enumerate(((xm, wr0_ref), (xp, wr1_ref))):
        a = jnp.concatenate([xb, sec], axis=-1).reshape(B * H * W, 2 * Cin)
        t = jnp.dot(a, w_ref[...], preferred_element_type=jnp.float32)
        t = t.reshape(B, H, W, 4 * C)
        t0 = t[..., 0 * C:1 * C]
        t1 = t[..., 1 * C:2 * C]
        t2 = t[..., 2 * C:3 * C]
        t3 = t[..., 3 * C:4 * C]
        e = t1 + jnp.concatenate([zcol, t3[:, :, :-1]], axis=2)
        o = t2 + jnp.concatenate([t0[:, :, 1:], zcol], axis=2)
        if pair:
            y = jnp.concatenate([e, o], axis=-1).astype(jnp.bfloat16)
            out_ref[:, :, r, :, :] = y
            yf = y.astype(jnp.float32).reshape(B * H * W, 2 * C)
            s2 = jnp.sum(yf, axis=0, keepdims=True)
            q2 = jnp.sum(yf * yf, axis=0, keepdims=True)
            sums.append(s2[:, :C] + s2[:, C:])
            sqs.append(q2[:, :C] + q2[:, C:])
        else:
            y = jnp.stack([e, o], axis=3).reshape(B, H, 2 * W, C)
            y = y.astype(jnp.bfloat16)
            out_ref[:, :, r, :, :] = y
            yf = y.astype(jnp.float32).reshape(B * H * 2 * W, C)
            sums.append(jnp.sum(yf, axis=0, keepdims=True))
            sqs.append(jnp.sum(yf * yf, axis=0, keepdims=True))
    mom_ref[0, 0, :] = (sums[0] + sums[1]).reshape(C)
    mom_ref[0, 1, :] = (sqs[0] + sqs[1]).reshape(C)


def _conv_layer(x, wr0, wr1, scale, shift, B, pair=False):
    N, H, W, Cin = x.shape
    C = wr0.shape[1] // 4
    G = N // B
    act = scale is not None

    in_specs = [
        pl.BlockSpec((B, H, W, Cin), lambda i: (i, 0, 0, 0)),
        pl.BlockSpec((2 * Cin, 4 * C), lambda i: (0, 0)),
        pl.BlockSpec((2 * Cin, 4 * C), lambda i: (0, 0)),
    ]
    args = [x, wr0, wr1]
    if act:
        in_specs += [
            pl.BlockSpec((1, Cin), lambda i: (0, 0)),
            pl.BlockSpec((1, Cin), lambda i: (0, 0)),
        ]
        args += [scale.reshape(1, Cin), shift.reshape(1, Cin)]

    if pair:
        oshape = (N, H, 2, W, 2 * C)
        oblock = (B, H, 2, W, 2 * C)
    else:
        oshape = (N, H, 2, 2 * W, C)
        oblock = (B, H, 2, 2 * W, C)

    y5, mom = pl.pallas_call(
        functools.partial(_layer_kernel, act=act, B=B, H=H, W=W, Cin=Cin,
                          C=C, pair=pair),
        out_shape=(
            jax.ShapeDtypeStruct(oshape, jnp.bfloat16),
            jax.ShapeDtypeStruct((G, 2, C), jnp.float32),
        ),
        grid_spec=pltpu.PrefetchScalarGridSpec(
            num_scalar_prefetch=0,
            grid=(G,),
            in_specs=in_specs,
            out_specs=(
                pl.BlockSpec(oblock, lambda i: (i, 0, 0, 0, 0)),
                pl.BlockSpec((1, 2, C), lambda i: (i, 0, 0)),
            ),
        ),
        compiler_params=pltpu.CompilerParams(
            dimension_semantics=("parallel",),
            vmem_limit_bytes=_VMEM_LIMIT,
        ),
    )(*args)
    return y5, mom


def _final_kernel(x_ref, w_ref, b_ref, sc_ref, sh_ref, out_ref, *, H, Wp, Cin):
    L = H * Wp
    xf = x_ref[0].astype(jnp.float32)
    xf = xf * sc_ref[...].reshape(1, 1, 2 * Cin) \
        + sh_ref[...].reshape(1, 1, 2 * Cin)
    xb = jnp.maximum(xf, 0.0).astype(jnp.bfloat16)
    t = lax.dot_general(w_ref[...], xb.reshape(L, 2 * Cin),
                        (((0,), (1,)), ((), ())),
                        preferred_element_type=jnp.float32)
    t = t.reshape(2, 16, 3, L)

    vv = lax.broadcasted_iota(jnp.int32, (3, L), 1) % Wp
    first = vv == 0
    last = vv == Wp - 1
    z1 = jnp.zeros((3, 1), jnp.float32)
    zW = jnp.zeros((3, Wp), jnp.float32)
    cls = [[], [], [], []]
    for kh in range(4):
        t_ = [[t[s, 4 * kh + i] for i in range(4)] for s in range(2)]
        shm = jnp.concatenate([z1, t_[1][3][:, :-1]], axis=1)
        cls[0].append(t_[0][1] + jnp.where(first, 0.0, shm))
        cls[1].append(t_[0][2] + t_[1][0])
        cls[2].append(t_[1][1] + t_[0][3])
        shp = jnp.concatenate([t_[0][0][:, 1:], z1], axis=1)
        cls[3].append(t_[1][2] + jnp.where(last, 0.0, shp))

    bias = b_ref[...].reshape(3, 1)
    for q in range(4):
        c_ = cls[q]
        v0 = c_[1] + jnp.concatenate([zW, c_[3][:, :-Wp]], axis=1)
        v1 = c_[2] + jnp.concatenate([c_[0][:, Wp:], zW], axis=1)
        out_ref[0, q] = jnp.tanh(v0 + bias).astype(jnp.bfloat16)
        out_ref[0, 4 + q] = jnp.tanh(v1 + bias).astype(jnp.bfloat16)


def _final_kernel_unused(x_ref, w_ref, b_ref, sc_ref, sh_ref, p_ref, out_ref,
                         xp_ref, *, H, Wp, Cin, RC):
    k = pl.program_id(1)

    @pl.when(k == 0)
    def _prologue():
        xf = x_ref[0].astype(jnp.float32)
        xf = xf * sc_ref[...].reshape(1, 1, 2 * Cin) \
            + sh_ref[...].reshape(1, 1, 2 * Cin)
        xp_ref[1:H + 1] = jnp.maximum(xf, 0.0).astype(jnp.bfloat16)
        z = jnp.zeros((1, Wp, 2 * Cin), jnp.bfloat16)
        xp_ref[0:1] = z
        xp_ref[H + 1:H + 2] = z

    KR = RC + 2
    L = KR * Wp
    xch = xp_ref[pl.ds(k * RC, KR)]
    t = lax.dot_general(w_ref[...], xch.reshape(L, 2 * Cin),
                        (((0,), (1,)), ((), ())),
                        preferred_element_type=jnp.float32)
    t = t.reshape(2, 16, 3, L)

    vv = lax.broadcasted_iota(jnp.int32, (3, L), 1) % Wp
    first = vv == 0
    last = vv == Wp - 1
    z1 = jnp.zeros((3, 1), jnp.float32)
    zW = jnp.zeros((3, Wp), jnp.float32)
    cls = [[], [], [], []]
    for kh in range(4):
        t_ = [[t[s, 4 * kh + i] for i in range(4)] for s in range(2)]
        shm = jnp.concatenate([z1, t_[1][3][:, :-1]], axis=1)
        cls[0].append(t_[0][1] + jnp.where(first, 0.0, shm))
        cls[1].append(t_[0][2] + t_[1][0])
        cls[2].append(t_[1][1] + t_[0][3])
        shp = jnp.concatenate([t_[0][0][:, 1:], z1], axis=1)
        cls[3].append(t_[1][2] + jnp.where(last, 0.0, shp))

    rows = []
    vq = []
    for q in range(4):
        c_ = cls[q]
        v0 = c_[1] + jnp.concatenate([zW, c_[3][:, :-Wp]], axis=1)
        v1 = c_[2] + jnp.concatenate([c_[0][:, Wp:], zW], axis=1)
        vq.append((v0, v1))
    for kk in range(1, KR - 1):
        for r in range(2):
            rows.append(jnp.concatenate(
                [vq[q][r][:, kk * Wp:(kk + 1) * Wp] for q in range(4)],
                axis=-1))
    g = jnp.stack(rows, axis=1).reshape(3 * 2 * RC, 4 * Wp)
    out = jnp.dot(g, p_ref[...], preferred_element_type=jnp.float32)
    out = jnp.tanh(out.reshape(3, 2 * RC, 4 * Wp) + b_ref[...].reshape(3, 1, 1))
    out_ref[0] = out


def _final_layer(x, bm4, b4, scale, shift):
    N, H, Wp, Cin2 = x.shape
    Cin = Cin2 // 2
    z = jnp.zeros((Cin, 48), bm4.dtype)
    w2 = jnp.concatenate(
        [jnp.concatenate([bm4, z], axis=1),
         jnp.concatenate([z, bm4], axis=1)], axis=0)
    sc2 = jnp.concatenate([scale, scale]).reshape(1, 2 * Cin)
    sh2 = jnp.concatenate([shift, shift]).reshape(1, 2 * Cin)
    o = pl.pallas_call(
        functools.partial(_final_kernel, H=H, Wp=Wp, Cin=Cin),
        out_shape=jax.ShapeDtypeStruct((N, 8, 3, H * Wp), jnp.bfloat16),
        grid_spec=pltpu.PrefetchScalarGridSpec(
            num_scalar_prefetch=0,
            grid=(N,),
            in_specs=[
                pl.BlockSpec((1, H, Wp, 2 * Cin), lambda i: (i, 0, 0, 0)),
                pl.BlockSpec((2 * Cin, 96), lambda i: (0, 0)),
                pl.BlockSpec((3, 1), lambda i: (0, 0)),
                pl.BlockSpec((1, 2 * Cin), lambda i: (0, 0)),
                pl.BlockSpec((1, 2 * Cin), lambda i: (0, 0)),
            ],
            out_specs=pl.BlockSpec((1, 8, 3, H * Wp), lambda i: (i, 0, 0, 0)),
        ),
        compiler_params=pltpu.CompilerParams(
            dimension_semantics=("parallel",),
            vmem_limit_bytes=_VMEM_LIMIT,
        ),
    )(x, w2, b4.reshape(3, 1).astype(jnp.float32), sc2, sh2)
    o = o.reshape(N, 2, 4, 3, H, Wp)
    o = jnp.transpose(o, (0, 3, 4, 1, 5, 2))
    return o.reshape(N, 3, 2 * H, 4 * Wp).astype(jnp.float32)


def _split_weights(bm):
    cin = bm.shape[0]
    c4 = bm.shape[1] // 4
    w = [bm[:, k * c4:(k + 1) * c4] for k in range(4)]
    wr0 = jnp.concatenate([w[1], w[3]], axis=0)
    wr1 = jnp.concatenate([w[2], w[0]], axis=0)
    return wr0, wr1


def _bn_stats(mom, count, gamma, beta):
    tot = jnp.sum(mom.astype(jnp.float32), axis=0)
    mean = tot[0] / count
    msq = tot[1] / count
    var = jnp.maximum(msq - jnp.square(mean), 0.0)
    scale = gamma * lax.rsqrt(var + 1e-5)
    shift = beta - mean * scale
    return scale, shift


def _pick_b(n, pref):
    b = min(pref, n)
    while b > 1 and (n % b or (n // b) % 2):
        b -= 1
    return b


@jax.jit
def _forward(x, params):
    N = x.shape[0]
    xb = jnp.transpose(x, (0, 2, 3, 1)).astype(jnp.bfloat16)

    scale = shift = None
    prefs = (32, 8, 4, 2)
    for li in range(4):
        bm, gamma, beta = params[li]
        wr0, wr1 = _split_weights(bm)
        B = _pick_b(N, prefs[li])
        pair = li == 3
        y5, mom = _conv_layer(xb, wr0, wr1, scale, shift, B, pair=pair)
        Nn, H, _, W2, C = y5.shape
        npix = Nn * 2 * H * W2 * (2 if pair else 1)
        scale, shift = _bn_stats(mom, jnp.float32(npix), gamma, beta)
        xb = y5.reshape(Nn, 2 * H, W2, C)

    bm4, b4 = params[4]
    return _final_layer(xb, bm4, b4, scale, shift)


def kernel(x, bm_0, b_0, gamma_0, beta_0, bm_1, b_1, gamma_1, beta_1,
           bm_2, b_2, gamma_2, beta_2, bm_3, b_3, gamma_3, beta_3,
           bm_4, b_4):
    params = [
        (bm_0, gamma_0, beta_0),
        (bm_1, gamma_1, beta_1),
        (bm_2, gamma_2, beta_2),
        (bm_3, gamma_3, beta_3),
        (bm_4, b_4),
    ]
    return _forward(x, params)


# speedup vs baseline: 3.4563x; 1.0621x over previous
import functools

import jax
import jax.numpy as jnp
from jax import lax
from jax.experimental import pallas as pl
from jax.experimental.pallas import tpu as pltpu

_VMEM_LIMIT = 48 * 1024 * 1024


def _layer_kernel(*refs, act, B, H, W, Cin, C, pair):
    if act:
        x_ref, wr0_ref, wr1_ref, sc_ref, sh_ref, out_ref, mom_ref = refs
        xf = x_ref[...].astype(jnp.float32)
        xf = xf * sc_ref[...].reshape(1, 1, 1, Cin) + sh_ref[...].reshape(1, 1, 1, Cin)
        xb = jnp.maximum(xf, 0.0).astype(jnp.bfloat16)
    else:
        x_ref, wr0_ref, wr1_ref, out_ref, mom_ref = refs
        xb = x_ref[...]

    zrow = jnp.zeros((B, 1, W, Cin), jnp.bfloat16)
    xm = jnp.concatenate([zrow, xb[:, :-1]], axis=1)
    xp = jnp.concatenate([xb[:, 1:], zrow], axis=1)

    zcol = jnp.zeros((B, H, 1, C), jnp.float32)
    sums = []
    sqs = []
    for r, (sec, w_ref) in enumerate(((xm, wr0_ref), (xp, wr1_ref))):
        a = jnp.concatenate([xb, sec], axis=-1).reshape(B * H * W, 2 * Cin)
        t = jnp.dot(a, w_ref[...], preferred_element_type=jnp.float32)
        t = t.reshape(B, H, W, 4 * C)
        t0 = t[..., 0 * C:1 * C]
        t1 = t[..., 1 * C:2 * C]
        t2 = t[..., 2 * C:3 * C]
        t3 = t[..., 3 * C:4 * C]
        e = t1 + jnp.concatenate([zcol, t3[:, :, :-1]], axis=2)
        o = t2 + jnp.concatenate([t0[:, :, 1:], zcol], axis=2)
        if pair:
            y = jnp.concatenate([e, o], axis=-1).astype(jnp.bfloat16)
            out_ref[:, :, r, :, :] = y
            yf = y.astype(jnp.float32).reshape(B * H * W, 2 * C)
            s2 = jnp.sum(yf, axis=0, keepdims=True)
            q2 = jnp.sum(yf * yf, axis=0, keepdims=True)
            sums.append(s2[:, :C] + s2[:, C:])
            sqs.append(q2[:, :C] + q2[:, C:])
        else:
            y = jnp.stack([e, o], axis=3).reshape(B, H, 2 * W, C)
            y = y.astype(jnp.bfloat16)
            out_ref[:, :, r, :, :] = y
            yf = y.astype(jnp.float32).reshape(B * H * 2 * W, C)
            sums.append(jnp.sum(yf, axis=0, keepdims=True))
            sqs.append(jnp.sum(yf * yf, axis=0, keepdims=True))
    mom_ref[0, 0, :] = (sums[0] + sums[1]).reshape(C)
    mom_ref[0, 1, :] = (sqs[0] + sqs[1]).reshape(C)


def _conv_layer(x, wr0, wr1, scale, shift, B, pair=False):
    N, H, W, Cin = x.shape
    C = wr0.shape[1] // 4
    G = N // B
    act = scale is not None

    in_specs = [
        pl.BlockSpec((B, H, W, Cin), lambda i: (i, 0, 0, 0)),
        pl.BlockSpec((2 * Cin, 4 * C), lambda i: (0, 0)),
        pl.BlockSpec((2 * Cin, 4 * C), lambda i: (0, 0)),
    ]
    args = [x, wr0, wr1]
    if act:
        in_specs += [
            pl.BlockSpec((1, Cin), lambda i: (0, 0)),
            pl.BlockSpec((1, Cin), lambda i: (0, 0)),
        ]
        args += [scale.reshape(1, Cin), shift.reshape(1, Cin)]

    if pair:
        oshape = (N, H, 2, W, 2 * C)
        oblock = (B, H, 2, W, 2 * C)
    else:
        oshape = (N, H, 2, 2 * W, C)
        oblock = (B, H, 2, 2 * W, C)

    y5, mom = pl.pallas_call(
        functools.partial(_layer_kernel, act=act, B=B, H=H, W=W, Cin=Cin,
                          C=C, pair=pair),
        out_shape=(
            jax.ShapeDtypeStruct(oshape, jnp.bfloat16),
            jax.ShapeDtypeStruct((G, 2, C), jnp.float32),
        ),
        grid_spec=pltpu.PrefetchScalarGridSpec(
            num_scalar_prefetch=0,
            grid=(G,),
            in_specs=in_specs,
            out_specs=(
                pl.BlockSpec(oblock, lambda i: (i, 0, 0, 0, 0)),
                pl.BlockSpec((1, 2, C), lambda i: (i, 0, 0)),
            ),
        ),
        compiler_params=pltpu.CompilerParams(
            dimension_semantics=("parallel",),
            vmem_limit_bytes=_VMEM_LIMIT,
        ),
    )(*args)
    return y5, mom


def _final_kernel(x_ref, w_ref, b_ref, sc_ref, sh_ref, out_ref,
                  *, B, H, Wp, Cin):
    L = H * Wp
    vv = lax.broadcasted_iota(jnp.int32, (3, L), 1) % Wp
    first = vv == 0
    last = vv == Wp - 1
    z1 = jnp.zeros((3, 1), jnp.float32)
    zW = jnp.zeros((3, Wp), jnp.float32)
    bias = b_ref[...].reshape(3, 1)
    for b in range(B):
        xf = x_ref[b].astype(jnp.float32)
        xf = xf * sc_ref[...].reshape(1, 1, 2 * Cin) \
            + sh_ref[...].reshape(1, 1, 2 * Cin)
        xb = jnp.maximum(xf, 0.0).astype(jnp.bfloat16)
        t = lax.dot_general(w_ref[...], xb.reshape(L, 2 * Cin),
                            (((0,), (1,)), ((), ())),
                            preferred_element_type=jnp.float32)
        t = t.reshape(2, 16, 3, L)
        cls = [[], [], [], []]
        for kh in range(4):
            t_ = [[t[s, 4 * kh + i] for i in range(4)] for s in range(2)]
            shm = jnp.concatenate([z1, t_[1][3][:, :-1]], axis=1)
            cls[0].append(t_[0][1] + jnp.where(first, 0.0, shm))
            cls[1].append(t_[0][2] + t_[1][0])
            cls[2].append(t_[1][1] + t_[0][3])
            shp = jnp.concatenate([t_[0][0][:, 1:], z1], axis=1)
            cls[3].append(t_[1][2] + jnp.where(last, 0.0, shp))
        for q in range(4):
            c_ = cls[q]
            v0 = c_[1] + jnp.concatenate([zW, c_[3][:, :-Wp]], axis=1)
            v1 = c_[2] + jnp.concatenate([c_[0][:, Wp:], zW], axis=1)
            out_ref[b, q] = jnp.tanh(v0 + bias).astype(jnp.bfloat16)
            out_ref[b, 4 + q] = jnp.tanh(v1 + bias).astype(jnp.bfloat16)


def _final_kernel_unused(x_ref, w_ref, b_ref, sc_ref, sh_ref, p_ref, out_ref,
                         xp_ref, *, H, Wp, Cin, RC):
    k = pl.program_id(1)

    @pl.when(k == 0)
    def _prologue():
        xf = x_ref[0].astype(jnp.float32)
        xf = xf * sc_ref[...].reshape(1, 1, 2 * Cin) \
            + sh_ref[...].reshape(1, 1, 2 * Cin)
        xp_ref[1:H + 1] = jnp.maximum(xf, 0.0).astype(jnp.bfloat16)
        z = jnp.zeros((1, Wp, 2 * Cin), jnp.bfloat16)
        xp_ref[0:1] = z
        xp_ref[H + 1:H + 2] = z

    KR = RC + 2
    L = KR * Wp
    xch = xp_ref[pl.ds(k * RC, KR)]
    t = lax.dot_general(w_ref[...], xch.reshape(L, 2 * Cin),
                        (((0,), (1,)), ((), ())),
                        preferred_element_type=jnp.float32)
    t = t.reshape(2, 16, 3, L)

    vv = lax.broadcasted_iota(jnp.int32, (3, L), 1) % Wp
    first = vv == 0
    last = vv == Wp - 1
    z1 = jnp.zeros((3, 1), jnp.float32)
    zW = jnp.zeros((3, Wp), jnp.float32)
    cls = [[], [], [], []]
    for kh in range(4):
        t_ = [[t[s, 4 * kh + i] for i in range(4)] for s in range(2)]
        shm = jnp.concatenate([z1, t_[1][3][:, :-1]], axis=1)
        cls[0].append(t_[0][1] + jnp.where(first, 0.0, shm))
        cls[1].append(t_[0][2] + t_[1][0])
        cls[2].append(t_[1][1] + t_[0][3])
        shp = jnp.concatenate([t_[0][0][:, 1:], z1], axis=1)
        cls[3].append(t_[1][2] + jnp.where(last, 0.0, shp))

    rows = []
    vq = []
    for q in range(4):
        c_ = cls[q]
        v0 = c_[1] + jnp.concatenate([zW, c_[3][:, :-Wp]], axis=1)
        v1 = c_[2] + jnp.concatenate([c_[0][:, Wp:], zW], axis=1)
        vq.append((v0, v1))
    for kk in range(1, KR - 1):
        for r in range(2):
            rows.append(jnp.concatenate(
                [vq[q][r][:, kk * Wp:(kk + 1) * Wp] for q in range(4)],
                axis=-1))
    g = jnp.stack(rows, axis=1).reshape(3 * 2 * RC, 4 * Wp)
    out = jnp.dot(g, p_ref[...], preferred_element_type=jnp.float32)
    out = jnp.tanh(out.reshape(3, 2 * RC, 4 * Wp) + b_ref[...].reshape(3, 1, 1))
    out_ref[0] = out


def _final_layer(x, bm4, b4, scale, shift):
    N, H, Wp, Cin2 = x.shape
    Cin = Cin2 // 2
    z = jnp.zeros((Cin, 48), bm4.dtype)
    w2 = jnp.concatenate(
        [jnp.concatenate([bm4, z], axis=1),
         jnp.concatenate([z, bm4], axis=1)], axis=0)
    sc2 = jnp.concatenate([scale, scale]).reshape(1, 2 * Cin)
    sh2 = jnp.concatenate([shift, shift]).reshape(1, 2 * Cin)
    B = 2 if N % 2 == 0 else 1
    o = pl.pallas_call(
        functools.partial(_final_kernel, B=B, H=H, Wp=Wp, Cin=Cin),
        out_shape=jax.ShapeDtypeStruct((N, 8, 3, H * Wp), jnp.bfloat16),
        grid_spec=pltpu.PrefetchScalarGridSpec(
            num_scalar_prefetch=0,
            grid=(N // B,),
            in_specs=[
                pl.BlockSpec((B, H, Wp, 2 * Cin), lambda i: (i, 0, 0, 0)),
                pl.BlockSpec((2 * Cin, 96), lambda i: (0, 0)),
                pl.BlockSpec((3, 1), lambda i: (0, 0)),
                pl.BlockSpec((1, 2 * Cin), lambda i: (0, 0)),
                pl.BlockSpec((1, 2 * Cin), lambda i: (0, 0)),
            ],
            out_specs=pl.BlockSpec((B, 8, 3, H * Wp), lambda i: (i, 0, 0, 0)),
        ),
        compiler_params=pltpu.CompilerParams(
            dimension_semantics=("parallel",),
            vmem_limit_bytes=_VMEM_LIMIT,
        ),
    )(x, w2, b4.reshape(3, 1).astype(jnp.float32), sc2, sh2)
    o = o.reshape(N, 2, 4, 3, H, Wp)
    o = jnp.transpose(o, (0, 3, 4, 1, 5, 2))
    return o.reshape(N, 3, 2 * H, 4 * Wp).astype(jnp.float32)


def _split_weights(bm):
    cin = bm.shape[0]
    c4 = bm.shape[1] // 4
    w = [bm[:, k * c4:(k + 1) * c4] for k in range(4)]
    wr0 = jnp.concatenate([w[1], w[3]], axis=0)
    wr1 = jnp.concatenate([w[2], w[0]], axis=0)
    return wr0, wr1


def _bn_stats(mom, count, gamma, beta):
    tot = jnp.sum(mom.astype(jnp.float32), axis=0)
    mean = tot[0] / count
    msq = tot[1] / count
    var = jnp.maximum(msq - jnp.square(mean), 0.0)
    scale = gamma * lax.rsqrt(var + 1e-5)
    shift = beta - mean * scale
    return scale, shift


def _pick_b(n, pref):
    b = min(pref, n)
    while b > 1 and (n % b or (n // b) % 2):
        b -= 1
    return b


@jax.jit
def _forward(x, params):
    N = x.shape[0]
    xb = jnp.transpose(x, (0, 2, 3, 1)).astype(jnp.bfloat16)

    scale = shift = None
    prefs = (32, 8, 4, 2)
    for li in range(4):
        bm, gamma, beta = params[li]
        wr0, wr1 = _split_weights(bm)
        B = _pick_b(N, prefs[li])
        pair = li == 3
        y5, mom = _conv_layer(xb, wr0, wr1, scale, shift, B, pair=pair)
        Nn, H, _, W2, C = y5.shape
        npix = Nn * 2 * H * W2 * (2 if pair else 1)
        scale, shift = _bn_stats(mom, jnp.float32(npix), gamma, beta)
        xb = y5.reshape(Nn, 2 * H, W2, C)

    bm4, b4 = params[4]
    return _final_layer(xb, bm4, b4, scale, shift)


def kernel(x, bm_0, b_0, gamma_0, beta_0, bm_1, b_1, gamma_1, beta_1,
           bm_2, b_2, gamma_2, beta_2, bm_3, b_3, gamma_3, beta_3,
           bm_4, b_4):
    params = [
        (bm_0, gamma_0, beta_0),
        (bm_1, gamma_1, beta_1),
        (bm_2, gamma_2, beta_2),
        (bm_3, gamma_3, beta_3),
        (bm_4, b_4),
    ]
    return _forward(x, params)
